# baseline (speedup 1.0000x reference)
import functools

import jax
import jax.numpy as jnp
from jax import lax
from jax.experimental import pallas as pl
from jax.experimental.pallas import tpu as pltpu
from jax.experimental.pallas import tpu_sc as plsc

_F0 = 1000000
_B = 16384
_D = 32
_L = 16
_FBS = 16000000


@functools.cache
def _make_sc_kernel(num_cores, num_subcores):
    nw = num_cores * num_subcores
    bpw = _B // nw
    nch = bpw // _L
    mesh = plsc.VectorSubcoreMesh(core_axis_name="c", subcore_axis_name="s")

    @functools.partial(
        pl.kernel,
        mesh=mesh,
        out_type=jax.ShapeDtypeStruct((_B,), jnp.float32),
        compiler_params=pltpu.CompilerParams(
            needs_layout_passes=False, use_tc_tiling_on_sc=False),
        scratch_types=[
            pltpu.VMEM((bpw,), jnp.int32),
            pltpu.VMEM((bpw,), jnp.int32),
            pltpu.VMEM((bpw,), jnp.int32),
            pltpu.VMEM((bpw * _D,), jnp.int32),
            pltpu.VMEM((bpw * _D,), jnp.int32),
            pltpu.VMEM((bpw * _D,), jnp.float32),
            pltpu.VMEM((bpw * _D,), jnp.float32),
            pltpu.VMEM((bpw,), jnp.float32),
            pltpu.VMEM((bpw,), jnp.float32),
            pltpu.VMEM((_D,), jnp.float32),
            pltpu.VMEM((_L,), jnp.float32),
            pltpu.VMEM((bpw,), jnp.float32),
            pltpu.SemaphoreType.DMA,
            pltpu.SemaphoreType.DMA,
            pltpu.SemaphoreType.DMA,
            pltpu.SemaphoreType.DMA,
        ],
    )
    def sc_kernel(uidx_hbm, iidx_hbm, iraw_hbm, tflat_hbm, ubias_hbm,
                  ibias_hbm, fcw_hbm, fcb_hbm, out_hbm,
                  uidx_v, iidx_v, iraw_v, uix_v, iix_v, urows_v, irows_v,
                  ubias_v, ibias_v, w_v, fcb_v, out_v,
                  sem_u, sem_i, sem_bu, sem_bi):
        wid = lax.axis_index("s") * num_cores + lax.axis_index("c")
        base = wid * bpw
        pltpu.sync_copy(uidx_hbm.at[pl.ds(base, bpw)], uidx_v)
        pltpu.sync_copy(iidx_hbm.at[pl.ds(base, bpw)], iidx_v)
        pltpu.sync_copy(iraw_hbm.at[pl.ds(base, bpw)], iraw_v)
        cbu = pltpu.async_copy(ubias_hbm.at[uidx_v], ubias_v, sem_bu)
        cbi = pltpu.async_copy(ibias_hbm.at[iraw_v], ibias_v, sem_bi)
        pltpu.sync_copy(fcw_hbm, w_v)
        pltpu.sync_copy(fcb_hbm, fcb_v)

        def build(j, carry):
            s = j * _L
            u = uidx_v[pl.ds(s, _L)]
            bu = u + (u >> 7) * 896
            i = iidx_v[pl.ds(s, _L)]
            bi = i + (i >> 7) * 896
            for d in range(_D):
                c = (d >> 3) * _FBS + (d & 7) * 128
                o = d * bpw + s
                uix_v[pl.ds(o, _L)] = bu + c
                iix_v[pl.ds(o, _L)] = bi + c
            return carry

        lax.fori_loop(0, nch, build, 0)
        cu = pltpu.async_copy(tflat_hbm.at[uix_v], urows_v, sem_u)
        ci = pltpu.async_copy(tflat_hbm.at[iix_v], irows_v, sem_i)
        cu.wait()
        ci.wait()
        cbu.wait()
        cbi.wait()

        w0v = w_v[pl.ds(0, _L)]
        w1v = w_v[pl.ds(_L, _L)]
        fbv = fcb_v[...]

        def body(g, carry):
            o = g * _L
            acc = fbv + ubias_v[pl.ds(o, _L)] + ibias_v[pl.ds(o, _L)]
            for d in range(_D):
                wd = w0v[d] if d < _L else w1v[d - _L]
                ud = urows_v[pl.ds(d * bpw + o, _L)]
                vd = irows_v[pl.ds(d * bpw + o, _L)]
                acc = acc + ud * vd * wd
            out_v[pl.ds(o, _L)] = acc
            return carry

        lax.fori_loop(0, nch, body, 0)
        pltpu.sync_copy(out_v, out_hbm.at[pl.ds(base, bpw)])

    return sc_kernel


def kernel(x, table, user_bias, item_bias, fc_w, fc_b):
    uidx = x[:, 0].astype(jnp.int32)
    iraw = x[:, 1].astype(jnp.int32)
    iidx = iraw + _F0
    tflat = jnp.transpose(
        jnp.reshape(table, (15625, 128, 4, 8)), (2, 0, 3, 1)).reshape(-1)
    info = plsc.get_sparse_core_info()
    k = _make_sc_kernel(info.num_cores, info.num_subcores)
    return k(uidx, iidx, iraw, tflat, user_bias.T.reshape(-1),
             item_bias.T.reshape(-1), fc_w.reshape(-1),
             jnp.broadcast_to(fc_b.reshape(()), (_L,)))


# speedup vs baseline: 1.3555x; 1.3555x over previous
import functools

import jax
import jax.numpy as jnp
from jax import lax
from jax.experimental import pallas as pl
from jax.experimental.pallas import tpu as pltpu
from jax.experimental.pallas import tpu_sc as plsc

_F0 = 1000000
_B = 16384
_D = 32
_L = 16
_FBS = 16000000


@functools.cache
def _make_dot_kernel(num_cores, num_subcores):
    nw = num_cores * num_subcores
    bpw = _B // nw
    nch = bpw // _L
    mesh = plsc.VectorSubcoreMesh(core_axis_name="c", subcore_axis_name="s")

    @functools.partial(
        pl.kernel,
        mesh=mesh,
        out_type=jax.ShapeDtypeStruct((_B,), jnp.float32),
        compiler_params=pltpu.CompilerParams(
            needs_layout_passes=False, use_tc_tiling_on_sc=False),
        scratch_types=[
            pltpu.VMEM((bpw,), jnp.int32),
            pltpu.VMEM((bpw,), jnp.int32),
            pltpu.VMEM((bpw * _D,), jnp.int32),
            pltpu.VMEM((bpw * _D,), jnp.int32),
            pltpu.VMEM((bpw * _D,), jnp.float32),
            pltpu.VMEM((bpw * _D,), jnp.float32),
            pltpu.VMEM((_D,), jnp.float32),
            pltpu.VMEM((_L,), jnp.float32),
            pltpu.VMEM((bpw,), jnp.float32),
            pltpu.SemaphoreType.DMA,
            pltpu.SemaphoreType.DMA,
        ],
    )
    def dot_kernel(uidx_hbm, iraw_hbm, tflat_hbm, fcw_hbm, fcb_hbm, out_hbm,
                   uidx_v, iraw_v, uix_v, iix_v, urows_v, irows_v,
                   w_v, fcb_v, out_v, sem_u, sem_i):
        wid = lax.axis_index("s") * num_cores + lax.axis_index("c")
        base = wid * bpw
        pltpu.sync_copy(uidx_hbm.at[pl.ds(base, bpw)], uidx_v)
        pltpu.sync_copy(iraw_hbm.at[pl.ds(base, bpw)], iraw_v)
        pltpu.sync_copy(fcw_hbm, w_v)
        pltpu.sync_copy(fcb_hbm, fcb_v)

        def build_u(j, carry):
            s = j * _L
            u = uidx_v[pl.ds(s, _L)]
            bu = u + (u >> 7) * 896
            for d in range(_D):
                c = (d >> 3) * _FBS + (d & 7) * 128
                uix_v[pl.ds(d * bpw + s, _L)] = bu + c
            return carry

        def build_i(j, carry):
            s = j * _L
            i = iraw_v[pl.ds(s, _L)] + _F0
            bi = i + (i >> 7) * 896
            for d in range(_D):
                c = (d >> 3) * _FBS + (d & 7) * 128
                iix_v[pl.ds(d * bpw + s, _L)] = bi + c
            return carry

        lax.fori_loop(0, nch, build_u, 0)
        cu = pltpu.async_copy(tflat_hbm.at[uix_v], urows_v, sem_u)
        lax.fori_loop(0, nch, build_i, 0)
        ci = pltpu.async_copy(tflat_hbm.at[iix_v], irows_v, sem_i)
        cu.wait()
        ci.wait()

        w0v = w_v[pl.ds(0, _L)]
        w1v = w_v[pl.ds(_L, _L)]
        fbv = fcb_v[...]

        def body(g, carry):
            o = g * _L
            acc = fbv
            for d in range(_D):
                wd = w0v[d] if d < _L else w1v[d - _L]
                ud = urows_v[pl.ds(d * bpw + o, _L)]
                vd = irows_v[pl.ds(d * bpw + o, _L)]
                acc = acc + ud * vd * wd
            out_v[pl.ds(o, _L)] = acc
            return carry

        lax.fori_loop(0, nch, body, 0)
        pltpu.sync_copy(out_v, out_hbm.at[pl.ds(base, bpw)])

    return dot_kernel


@functools.cache
def _make_bias_kernel(num_cores, num_subcores):
    nw = num_cores * num_subcores
    bpw = _B // nw
    nch = bpw // _L
    mesh = plsc.VectorSubcoreMesh(core_axis_name="c", subcore_axis_name="s")

    @functools.partial(
        pl.kernel,
        mesh=mesh,
        out_type=jax.ShapeDtypeStruct((_B,), jnp.float32),
        compiler_params=pltpu.CompilerParams(
            needs_layout_passes=False, use_tc_tiling_on_sc=False),
        scratch_types=[
            pltpu.VMEM((bpw,), jnp.int32),
            pltpu.VMEM((bpw,), jnp.int32),
            pltpu.VMEM((bpw,), jnp.float32),
            pltpu.VMEM((bpw,), jnp.float32),
            pltpu.VMEM((bpw,), jnp.float32),
            pltpu.SemaphoreType.DMA,
            pltpu.SemaphoreType.DMA,
        ],
    )
    def bias_kernel(uidx_hbm, iraw_hbm, ubias_hbm, ibias_hbm, part_hbm,
                    out_hbm, uidx_v, iraw_v, ubias_v, ibias_v, part_v,
                    sem_bu, sem_bi):
        wid = lax.axis_index("s") * num_cores + lax.axis_index("c")
        base = wid * bpw
        pltpu.sync_copy(uidx_hbm.at[pl.ds(base, bpw)], uidx_v)
        pltpu.sync_copy(iraw_hbm.at[pl.ds(base, bpw)], iraw_v)
        cbu = pltpu.async_copy(ubias_hbm.at[uidx_v], ubias_v, sem_bu)
        cbi = pltpu.async_copy(ibias_hbm.at[iraw_v], ibias_v, sem_bi)
        pltpu.sync_copy(part_hbm.at[pl.ds(base, bpw)], part_v)
        cbu.wait()
        cbi.wait()

        def body(g, carry):
            o = g * _L
            acc = (part_v[pl.ds(o, _L)] + ubias_v[pl.ds(o, _L)]
                   + ibias_v[pl.ds(o, _L)])
            part_v[pl.ds(o, _L)] = acc
            return carry

        lax.fori_loop(0, nch, body, 0)
        pltpu.sync_copy(part_v, out_hbm.at[pl.ds(base, bpw)])

    return bias_kernel


def kernel(x, table, user_bias, item_bias, fc_w, fc_b):
    uidx = x[:, 0].astype(jnp.int32)
    iraw = x[:, 1].astype(jnp.int32)
    tflat = jnp.transpose(
        jnp.reshape(table, (15625, 128, 4, 8)), (2, 0, 3, 1)).reshape(-1)
    info = plsc.get_sparse_core_info()
    kd = _make_dot_kernel(info.num_cores, info.num_subcores)
    kb = _make_bias_kernel(info.num_cores, info.num_subcores)
    part = kd(uidx, iraw, tflat, fc_w.reshape(-1),
              jnp.broadcast_to(fc_b.reshape(()), (_L,)))
    return kb(uidx, iraw, user_bias.reshape(-1), item_bias.reshape(-1), part)
